# baseline (speedup 1.0000x reference)
import functools

import jax
import jax.numpy as jnp
from jax import lax
from jax.experimental import pallas as pl
from jax.experimental.pallas import tpu as pltpu
from jax.experimental.pallas import tpu_sc as plsc

NUM_EXPERTS = 64
NUM_GROUPS = 8
GROUP_SIZE = NUM_EXPERTS // NUM_GROUPS
TOP_GROUPS = 4
TOPK = 8
TOKEN_BLOCK = 1024

_NEG = float("-inf")

_NC = 2
_NS = 16
_NW = _NC * _NS


def _router_kernel(x_ref, wt_ref, b_ref, ts_ref, idx_ref):
    x = x_ref[...]
    wt = wt_ref[...]
    lt = jax.lax.dot_general(
        wt, x, (((0,), (1,)), ((), ())), preferred_element_type=jnp.float32
    )
    s = jax.nn.sigmoid(lt)
    sfc = s + b_ref[...]
    tb = s.shape[1]

    li8 = jax.lax.broadcasted_iota(jnp.int32, (GROUP_SIZE, tb), 0)
    gs_rows = []
    for g in range(NUM_GROUPS):
        slab = sfc[g * GROUP_SIZE:(g + 1) * GROUP_SIZE, :]
        m1 = jnp.max(slab, axis=0, keepdims=True)
        i1 = jnp.min(jnp.where(slab == m1, li8, GROUP_SIZE), axis=0,
                     keepdims=True)
        m2 = jnp.max(jnp.where(li8 == i1, _NEG, slab), axis=0, keepdims=True)
        gs_rows.append(m1 + m2)
    gs = jnp.concatenate(gs_rows, axis=0)

    iota_g = jax.lax.broadcasted_iota(jnp.int32, (NUM_GROUPS, tb), 0)
    keep = jnp.zeros((NUM_GROUPS, tb), jnp.bool_)
    gm = gs
    for _ in range(TOP_GROUPS):
        mg = jnp.max(gm, axis=0, keepdims=True)
        gi = jnp.min(jnp.where(gm == mg, iota_g, NUM_GROUPS), axis=0,
                     keepdims=True)
        hit_g = iota_g == gi
        keep = keep | hit_g
        gm = jnp.where(hit_g, _NEG, gm)

    masked = jnp.concatenate(
        [jnp.where(keep[g:g + 1, :], sfc[g * GROUP_SIZE:(g + 1) * GROUP_SIZE, :], _NEG)
         for g in range(NUM_GROUPS)], axis=0)

    iota_e = jax.lax.broadcasted_iota(jnp.int32, (NUM_EXPERTS, tb), 0)
    vals, idxs = [], []
    for _ in range(TOPK):
        m = jnp.max(masked, axis=0, keepdims=True)
        e = jnp.min(jnp.where(masked == m, iota_e, NUM_EXPERTS), axis=0,
                    keepdims=True)
        hit = iota_e == e
        vals.append(jnp.sum(jnp.where(hit, s, 0.0), axis=0, keepdims=True))
        idxs.append(e)
        masked = jnp.where(hit, _NEG, masked)
    vt = jnp.concatenate(vals, axis=0)
    it = jnp.concatenate(idxs, axis=0)

    denom = jnp.sum(vt, axis=0, keepdims=True) + 1e-20
    ts_ref[...] = (vt / denom).T
    idx_ref[...] = it.T


def _hist_body(idx_hbm, out_hbm, chunk, bins, stage):
    c = lax.axis_index("c")
    s = lax.axis_index("s")
    wid = s * _NC + c
    n_idx = idx_hbm.shape[0]
    chunk_len = n_idx // _NW
    pltpu.sync_copy(idx_hbm.at[pl.ds(wid * chunk_len, chunk_len)], chunk)
    zeros16 = jnp.zeros((16,), jnp.int32)
    for j in range(8):
        bins[pl.ds(j * 16, 16)] = zeros16
    ones16 = jnp.full((16,), 1, jnp.int32)
    off = jnp.where(jax.lax.broadcasted_iota(jnp.int32, (16,), 0) >= 8,
                    NUM_EXPERTS, 0)

    def body(i, carry):
        v = chunk[pl.ds(i * 16, 16)] + off
        plsc.addupdate_scatter(bins, [v], ones16)
        return carry

    jax.lax.fori_loop(0, chunk_len // 16, body, 0)

    for j in range(4):
        stage[pl.ds(j * 16, 16)] = (bins[pl.ds(j * 16, 16)]
                                    + bins[pl.ds(NUM_EXPERTS + j * 16, 16)])
    pltpu.sync_copy(stage, out_hbm.at[wid])


_hist_kernel = functools.partial(
    pl.kernel,
    out_type=jax.ShapeDtypeStruct((_NW, NUM_EXPERTS), jnp.int32),
    mesh=plsc.VectorSubcoreMesh(core_axis_name="c", subcore_axis_name="s"),
    scratch_types=[
        pltpu.VMEM((32768 * TOPK // _NW,), jnp.int32),
        pltpu.VMEM((2 * NUM_EXPERTS,), jnp.int32),
        pltpu.VMEM((NUM_EXPERTS,), jnp.int32),
    ],
    compiler_params=pltpu.CompilerParams(needs_layout_passes=False),
)(_hist_body)


@functools.partial(jax.jit, static_argnames=())
def kernel(x, expert_bias, W):
    n, dim = x.shape
    wt = W.T
    b = expert_bias.reshape(NUM_EXPERTS, 1)
    grid = (n // TOKEN_BLOCK,)
    ts, idx = pl.pallas_call(
        _router_kernel,
        grid=grid,
        in_specs=[
            pl.BlockSpec((TOKEN_BLOCK, dim), lambda i: (i, 0)),
            pl.BlockSpec((dim, NUM_EXPERTS), lambda i: (0, 0)),
            pl.BlockSpec((NUM_EXPERTS, 1), lambda i: (0, 0)),
        ],
        out_specs=[
            pl.BlockSpec((TOKEN_BLOCK, TOPK), lambda i: (i, 0)),
            pl.BlockSpec((TOKEN_BLOCK, TOPK), lambda i: (i, 0)),
        ],
        out_shape=[
            jax.ShapeDtypeStruct((n, TOPK), jnp.float32),
            jax.ShapeDtypeStruct((n, TOPK), jnp.int32),
        ],
        compiler_params=pltpu.CompilerParams(
            dimension_semantics=("arbitrary",),
        ),
    )(x, wt, b)
    partials = _hist_kernel(idx.reshape(-1))
    cnt = jnp.sum(partials, axis=0)
    return ts, idx, cnt


# speedup vs baseline: 1.1319x; 1.1319x over previous
import functools

import jax
import jax.numpy as jnp
from jax.experimental import pallas as pl
from jax.experimental.pallas import tpu as pltpu

NUM_EXPERTS = 64
NUM_GROUPS = 8
GROUP_SIZE = NUM_EXPERTS // NUM_GROUPS
TOP_GROUPS = 4
TOPK = 8
TOKEN_BLOCK = 1024

_NEG = float("-inf")


def _router_kernel(x1_ref, x2_ref, wt_ref, b_ref, ts_ref, idx_ref, cnt_ref):
    i = pl.program_id(0)
    x1 = x1_ref[...]
    x2 = x2_ref[...]
    wt = wt_ref[...]
    h = wt.shape[0] // 2
    lt = jax.lax.dot_general(
        wt[:h], x1, (((0,), (1,)), ((), ())), preferred_element_type=jnp.float32
    ) + jax.lax.dot_general(
        wt[h:], x2, (((0,), (1,)), ((), ())), preferred_element_type=jnp.float32
    )
    s = jax.nn.sigmoid(lt)
    sfc = s + b_ref[...]
    tb = s.shape[1]

    li8 = jax.lax.broadcasted_iota(jnp.int32, (GROUP_SIZE, tb), 0)
    gs_rows = []
    for g in range(NUM_GROUPS):
        slab = sfc[g * GROUP_SIZE:(g + 1) * GROUP_SIZE, :]
        m1 = jnp.max(slab, axis=0, keepdims=True)
        i1 = jnp.min(jnp.where(slab == m1, li8, GROUP_SIZE), axis=0,
                     keepdims=True)
        m2 = jnp.max(jnp.where(li8 == i1, _NEG, slab), axis=0, keepdims=True)
        gs_rows.append(m1 + m2)
    gs = jnp.concatenate(gs_rows, axis=0)

    iota_g = jax.lax.broadcasted_iota(jnp.int32, (NUM_GROUPS, tb), 0)
    keep = jnp.zeros((NUM_GROUPS, tb), jnp.bool_)
    gm = gs
    for _ in range(TOP_GROUPS):
        mg = jnp.max(gm, axis=0, keepdims=True)
        gi = jnp.min(jnp.where(gm == mg, iota_g, NUM_GROUPS), axis=0,
                     keepdims=True)
        hit_g = iota_g == gi
        keep = keep | hit_g
        gm = jnp.where(hit_g, _NEG, gm)

    masked = jnp.concatenate(
        [jnp.where(keep[g:g + 1, :], sfc[g * GROUP_SIZE:(g + 1) * GROUP_SIZE, :], _NEG)
         for g in range(NUM_GROUPS)], axis=0)

    iota_e = jax.lax.broadcasted_iota(jnp.int32, (NUM_EXPERTS, tb), 0)
    vals, idxs = [], []
    selcnt = jnp.zeros((NUM_EXPERTS, tb), jnp.float32)
    for _ in range(TOPK):
        m = jnp.max(masked, axis=0, keepdims=True)
        e = jnp.min(jnp.where(masked == m, iota_e, NUM_EXPERTS), axis=0,
                    keepdims=True)
        hit = iota_e == e
        vals.append(jnp.sum(jnp.where(hit, s, 0.0), axis=0, keepdims=True))
        idxs.append(e)
        selcnt = selcnt + hit.astype(jnp.float32)
        masked = jnp.where(hit, _NEG, masked)
    vt = jnp.concatenate(vals, axis=0)
    it = jnp.concatenate(idxs, axis=0)

    denom = jnp.sum(vt, axis=0, keepdims=True) + 1e-20
    ts_ref[...] = (vt / denom).T
    idx_ref[...] = it.T

    blk_cnt = jnp.sum(selcnt, axis=1, keepdims=True).astype(jnp.int32)

    @pl.when(i == 0)
    def _():
        cnt_ref[...] = blk_cnt

    @pl.when(i != 0)
    def _():
        cnt_ref[...] = cnt_ref[...] + blk_cnt


@functools.partial(jax.jit, static_argnames=())
def kernel(x, expert_bias, W):
    n, dim = x.shape
    wt = W.T
    b = expert_bias.reshape(NUM_EXPERTS, 1)
    grid = (n // TOKEN_BLOCK,)
    ts, idx, cnt = pl.pallas_call(
        _router_kernel,
        grid=grid,
        in_specs=[
            pl.BlockSpec((TOKEN_BLOCK, dim // 2), lambda i: (i, 0)),
            pl.BlockSpec((TOKEN_BLOCK, dim // 2), lambda i: (i, 1)),
            pl.BlockSpec((dim, NUM_EXPERTS), lambda i: (0, 0)),
            pl.BlockSpec((NUM_EXPERTS, 1), lambda i: (0, 0)),
        ],
        out_specs=[
            pl.BlockSpec((TOKEN_BLOCK, TOPK), lambda i: (i, 0)),
            pl.BlockSpec((TOKEN_BLOCK, TOPK), lambda i: (i, 0)),
            pl.BlockSpec((NUM_EXPERTS, 1), lambda i: (0, 0)),
        ],
        out_shape=[
            jax.ShapeDtypeStruct((n, TOPK), jnp.float32),
            jax.ShapeDtypeStruct((n, TOPK), jnp.int32),
            jax.ShapeDtypeStruct((NUM_EXPERTS, 1), jnp.int32),
        ],
        compiler_params=pltpu.CompilerParams(
            dimension_semantics=("arbitrary",),
        ),
    )(x, x, wt, b)
    return ts, idx, cnt.reshape(NUM_EXPERTS)


# speedup vs baseline: 1.1527x; 1.0184x over previous
import functools

import jax
import jax.numpy as jnp
from jax.experimental import pallas as pl
from jax.experimental.pallas import tpu as pltpu

NUM_EXPERTS = 64
NUM_GROUPS = 8
GROUP_SIZE = NUM_EXPERTS // NUM_GROUPS
TOP_GROUPS = 4
TOPK = 8
TOKEN_BLOCK = 1024

_NEG = float("-inf")


def _router_kernel(x_ref, w_ref, b_ref, ts_ref, idx_ref, cnt_ref):
    i = pl.program_id(0)
    x = x_ref[...]
    w = w_ref[...]
    lt = jax.lax.dot_general(
        x, w, (((1,), (1,)), ((), ())), preferred_element_type=jnp.float32
    ).T
    s = jax.nn.sigmoid(lt)
    sfc = s + b_ref[...]
    tb = s.shape[1]

    li8 = jax.lax.broadcasted_iota(jnp.int32, (GROUP_SIZE, tb), 0)
    gs_rows = []
    for g in range(NUM_GROUPS):
        slab = sfc[g * GROUP_SIZE:(g + 1) * GROUP_SIZE, :]
        m1 = jnp.max(slab, axis=0, keepdims=True)
        i1 = jnp.min(jnp.where(slab == m1, li8, GROUP_SIZE), axis=0,
                     keepdims=True)
        m2 = jnp.max(jnp.where(li8 == i1, _NEG, slab), axis=0, keepdims=True)
        gs_rows.append(m1 + m2)
    gs = jnp.concatenate(gs_rows, axis=0)

    iota_g = jax.lax.broadcasted_iota(jnp.int32, (NUM_GROUPS, tb), 0)
    keep = jnp.zeros((NUM_GROUPS, tb), jnp.bool_)
    gm = gs
    for _ in range(TOP_GROUPS):
        mg = jnp.max(gm, axis=0, keepdims=True)
        gi = jnp.min(jnp.where(gm == mg, iota_g, NUM_GROUPS), axis=0,
                     keepdims=True)
        hit_g = iota_g == gi
        keep = keep | hit_g
        gm = jnp.where(hit_g, _NEG, gm)

    masked = jnp.concatenate(
        [jnp.where(keep[g:g + 1, :], sfc[g * GROUP_SIZE:(g + 1) * GROUP_SIZE, :], _NEG)
         for g in range(NUM_GROUPS)], axis=0)

    iota_e = jax.lax.broadcasted_iota(jnp.int32, (NUM_EXPERTS, tb), 0)
    vals, idxs = [], []
    selcnt = jnp.zeros((NUM_EXPERTS, tb), jnp.float32)
    for _ in range(TOPK):
        m = jnp.max(masked, axis=0, keepdims=True)
        e = jnp.min(jnp.where(masked == m, iota_e, NUM_EXPERTS), axis=0,
                    keepdims=True)
        hit = iota_e == e
        vals.append(jnp.sum(jnp.where(hit, s, 0.0), axis=0, keepdims=True))
        idxs.append(e)
        selcnt = selcnt + hit.astype(jnp.float32)
        masked = jnp.where(hit, _NEG, masked)
    vt = jnp.concatenate(vals, axis=0)
    it = jnp.concatenate(idxs, axis=0)

    denom = jnp.sum(vt, axis=0, keepdims=True) + 1e-20
    ts_ref[...] = (vt / denom).T
    idx_ref[...] = it.T

    blk_cnt = jnp.sum(selcnt, axis=1, keepdims=True).astype(jnp.int32)

    @pl.when(i == 0)
    def _():
        cnt_ref[...] = blk_cnt

    @pl.when(i != 0)
    def _():
        cnt_ref[...] = cnt_ref[...] + blk_cnt


@functools.partial(jax.jit, static_argnames=())
def kernel(x, expert_bias, W):
    n, dim = x.shape
    b = expert_bias.reshape(NUM_EXPERTS, 1)
    grid = (n // TOKEN_BLOCK,)
    ts, idx, cnt = pl.pallas_call(
        _router_kernel,
        grid=grid,
        in_specs=[
            pl.BlockSpec((TOKEN_BLOCK, dim), lambda i: (i, 0)),
            pl.BlockSpec((NUM_EXPERTS, dim), lambda i: (0, 0)),
            pl.BlockSpec((NUM_EXPERTS, 1), lambda i: (0, 0)),
        ],
        out_specs=[
            pl.BlockSpec((TOKEN_BLOCK, TOPK), lambda i: (i, 0)),
            pl.BlockSpec((TOKEN_BLOCK, TOPK), lambda i: (i, 0)),
            pl.BlockSpec((NUM_EXPERTS, 1), lambda i: (0, 0)),
        ],
        out_shape=[
            jax.ShapeDtypeStruct((n, TOPK), jnp.float32),
            jax.ShapeDtypeStruct((n, TOPK), jnp.int32),
            jax.ShapeDtypeStruct((NUM_EXPERTS, 1), jnp.int32),
        ],
        compiler_params=pltpu.CompilerParams(
            dimension_semantics=("arbitrary",),
        ),
    )(x, W, b)
    return ts, idx, cnt.reshape(NUM_EXPERTS)
